# Initial kernel scaffold; baseline (speedup 1.0000x reference)
import jax
import jax.numpy as jnp
from jax.experimental import pallas as pl


def kernel(h, r, t, entity_emb, relation_emb):
    raise NotImplementedError("write your pallas kernel here")


# speedup vs baseline: 1.9901x; 1.9901x over previous
import jax
import jax.numpy as jnp
from jax import lax
from jax.experimental import pallas as pl
from jax.experimental.pallas import tpu as pltpu
from jax.experimental.pallas import tpu_sc as plsc

NC = 2
NS = 16
NW = NC * NS
L = 16
D = 128
G = 128


def _sqrt16(s):
    i = plsc.bitcast(s, jnp.int32)
    i = jnp.int32(0x5F3759DF) - (i >> 1)
    y = plsc.bitcast(i, jnp.float32)
    half_s = s * jnp.float32(0.5)
    for _ in range(3):
        y = y * (jnp.float32(1.5) - half_s * y * y)
    return s * y


def _lane_sum(acc, lanes):
    for k in (8, 4, 2, 1):
        perm = lanes ^ k
        acc = acc + jnp.take_along_axis(acc, perm, axis=0)
    return acc


def _make_body(bpw, nchunk):
    def body(h_hbm, r_hbm, t_hbm, ent_hbm, rel_hbm, out_hbm,
             idx_h, idx_r, idx_t, rows_h, rows_r, rows_t, out_v, sem):
        wid = lax.axis_index("s") * NC + lax.axis_index("c")
        base = wid * bpw
        lanes = lax.iota(jnp.int32, L)
        for c in range(nchunk):
            pltpu.sync_copy(h_hbm.at[pl.ds(base + c * G, G)], idx_h.at[c])
            pltpu.sync_copy(r_hbm.at[pl.ds(base + c * G, G)], idx_r.at[c])
            pltpu.sync_copy(t_hbm.at[pl.ds(base + c * G, G)], idx_t.at[c])
            ch = pltpu.async_copy(ent_hbm.at[idx_h.at[c]], rows_h, sem)
            cr = pltpu.async_copy(rel_hbm.at[idx_r.at[c]], rows_r, sem)
            ct = pltpu.async_copy(ent_hbm.at[idx_t.at[c]], rows_t, sem)
            ch.wait()
            cr.wait()
            ct.wait()

            def group(g, _, c=c):
                def elem(i, res, g=g):
                    row = g * L + i
                    acc = jnp.zeros((L,), jnp.float32)
                    for k in range(D // L):
                        hv = rows_h[row, pl.ds(k * L, L)]
                        rv = rows_r[row, pl.ds(k * L, L)]
                        tv = rows_t[row, pl.ds(k * L, L)]
                        dv = hv + rv - tv
                        acc = acc + dv * dv
                    tot = _lane_sum(acc, lanes)
                    return jnp.where(lanes == i, tot, res)

                res = lax.fori_loop(0, L, elem, jnp.zeros((L,), jnp.float32))
                out_v[pl.ds(c * G + g * L, L)] = _sqrt16(res)
                return _

            lax.fori_loop(0, G // L, group, 0)
        pltpu.sync_copy(out_v, out_hbm.at[pl.ds(base, bpw)])
    return body


def kernel(h, r, t, entity_emb, relation_emb):
    b = h.shape[0]
    assert b % (NW * G) == 0
    bpw = b // NW
    nchunk = bpw // G
    mesh = plsc.VectorSubcoreMesh(core_axis_name="c", subcore_axis_name="s",
                                  num_cores=NC, num_subcores=NS)
    f = pl.kernel(
        _make_body(bpw, nchunk),
        out_type=jax.ShapeDtypeStruct((b,), jnp.float32),
        mesh=mesh,
        compiler_params=pltpu.CompilerParams(needs_layout_passes=False),
        scratch_types=[
            pltpu.VMEM((nchunk, G), jnp.int32),
            pltpu.VMEM((nchunk, G), jnp.int32),
            pltpu.VMEM((nchunk, G), jnp.int32),
            pltpu.VMEM((G, D), jnp.float32),
            pltpu.VMEM((G, D), jnp.float32),
            pltpu.VMEM((G, D), jnp.float32),
            pltpu.VMEM((bpw,), jnp.float32),
            pltpu.SemaphoreType.DMA,
        ],
    )
    return f(h.astype(jnp.int32), r.astype(jnp.int32), t.astype(jnp.int32),
             entity_emb, relation_emb)


# speedup vs baseline: 2.5128x; 1.2627x over previous
import jax
import jax.numpy as jnp
from jax import lax
from jax.experimental import pallas as pl
from jax.experimental.pallas import tpu as pltpu
from jax.experimental.pallas import tpu_sc as plsc

NC = 2
NS = 16
NW = NC * NS
L = 16
D = 128
G = 128


def _sqrt16(s):
    i = plsc.bitcast(s, jnp.int32)
    i = jnp.int32(0x5F3759DF) - (i >> 1)
    y = plsc.bitcast(i, jnp.float32)
    half_s = s * jnp.float32(0.5)
    for _ in range(3):
        y = y * (jnp.float32(1.5) - half_s * y * y)
    return s * y


def _perm(x, idx):
    return jnp.take_along_axis(x, idx, axis=0)


def _merge(x, y, k, lanes):
    m = (lanes & k) == 0
    perm = lanes ^ k
    return jnp.where(m, x, _perm(y, perm)) + jnp.where(m, _perm(x, perm), y)


def _make_body(bpw, nchunk):
    def body(h_hbm, r_hbm, t_hbm, ent_hbm, rel_hbm, out_hbm,
             idx_h, idx_r, idx_t,
             hr0, hr1, hr2, t0, t1, t2, out_v,
             semi, srel, sadd, st0, st1, st2):
        wid = lax.axis_index("s") * NC + lax.axis_index("c")
        base = wid * bpw
        lanes = lax.iota(jnp.int32, L)

        stg = []
        for c in range(nchunk):
            stg.append(pltpu.async_copy(
                h_hbm.at[pl.ds(base + c * G, G)], idx_h.at[c], semi))
            stg.append(pltpu.async_copy(
                r_hbm.at[pl.ds(base + c * G, G)], idx_r.at[c], semi))
            stg.append(pltpu.async_copy(
                t_hbm.at[pl.ds(base + c * G, G)], idx_t.at[c], semi))
        for d in stg:
            d.wait()

        hrbufs = (hr0, hr1, hr2)
        tbufs = (t0, t1, t2)
        tsems = (st0, st1, st2)

        def fire_rel(c):
            return pltpu.async_copy(rel_hbm.at[idx_r.at[c]],
                                    hrbufs[c % 3], srel)

        def fire_add(c):
            return pltpu.async_copy(ent_hbm.at[idx_h.at[c]],
                                    hrbufs[c % 3], sadd, add=True)

        def fire_t(c):
            return pltpu.async_copy(ent_hbm.at[idx_t.at[c]],
                                    tbufs[c % 3], tsems[c % 3])

        d_rel = {0: fire_rel(0)}
        d_t = {0: fire_t(0)}
        d_rel[1] = fire_rel(1)
        d_t[1] = fire_t(1)
        d_rel[0].wait()
        d_add = {0: fire_add(0)}

        for c in range(nchunk):
            d_add[c].wait()
            d_t[c].wait()
            if c + 1 < nchunk:
                d_rel[c + 1].wait()
                d_add[c + 1] = fire_add(c + 1)
            if c + 2 < nchunk:
                d_rel[c + 2] = fire_rel(c + 2)
                d_t[c + 2] = fire_t(c + 2)
            rhr = hrbufs[c % 3]
            rt = tbufs[c % 3]

            def group(g, f_prev, c=c, rhr=rhr, rt=rt):
                def one_acc(row):
                    acc = jnp.zeros((L,), jnp.float32)
                    for k in range(D // L):
                        hv = rhr[row, pl.ds(k * L, L)]
                        tv = rt[row, pl.ds(k * L, L)]
                        dv = hv - tv
                        acc = acc + dv * dv
                    return acc

                perm8 = lanes ^ 8
                vs = []
                for p in range(8):
                    a = one_acc(g * 8 + p)
                    vs.append(a + _perm(a, perm8))
                for k in (4, 2, 1):
                    half = len(vs) // 2
                    vs = [_merge(x, y, k, lanes)
                          for x, y in zip(vs[:half], vs[half:])]
                f = vs[0]

                @pl.when(g % 2 == 1)
                def _():
                    res = jnp.where(lanes < 8, f_prev, f)
                    out_v[pl.ds(c * G + (g // 2) * L, L)] = _sqrt16(res)

                return f

            lax.fori_loop(0, G // 8, group, jnp.zeros((L,), jnp.float32))
        pltpu.sync_copy(out_v, out_hbm.at[pl.ds(base, bpw)])
    return body


def kernel(h, r, t, entity_emb, relation_emb):
    b = h.shape[0]
    assert b % (NW * G) == 0
    bpw = b // NW
    nchunk = bpw // G
    mesh = plsc.VectorSubcoreMesh(core_axis_name="c", subcore_axis_name="s",
                                  num_cores=NC, num_subcores=NS)
    f = pl.kernel(
        _make_body(bpw, nchunk),
        out_type=jax.ShapeDtypeStruct((b,), jnp.float32),
        mesh=mesh,
        compiler_params=pltpu.CompilerParams(
            needs_layout_passes=False,
            disable_bounds_checks=True,
            disable_semaphore_checks=True,
            skip_device_barrier=True,
        ),
        scratch_types=[
            pltpu.VMEM((nchunk, G), jnp.int32),
            pltpu.VMEM((nchunk, G), jnp.int32),
            pltpu.VMEM((nchunk, G), jnp.int32),
            pltpu.VMEM((G, D), jnp.float32),
            pltpu.VMEM((G, D), jnp.float32),
            pltpu.VMEM((G, D), jnp.float32),
            pltpu.VMEM((G, D), jnp.float32),
            pltpu.VMEM((G, D), jnp.float32),
            pltpu.VMEM((G, D), jnp.float32),
            pltpu.VMEM((bpw,), jnp.float32),
            pltpu.SemaphoreType.DMA,
            pltpu.SemaphoreType.DMA,
            pltpu.SemaphoreType.DMA,
            pltpu.SemaphoreType.DMA,
            pltpu.SemaphoreType.DMA,
            pltpu.SemaphoreType.DMA,
        ],
    )
    return f(h.astype(jnp.int32), r.astype(jnp.int32), t.astype(jnp.int32),
             entity_emb, relation_emb)


# speedup vs baseline: 2.7359x; 1.0888x over previous
import jax
import jax.numpy as jnp
from jax import lax
from jax.experimental import pallas as pl
from jax.experimental.pallas import tpu as pltpu
from jax.experimental.pallas import tpu_sc as plsc

NC = 2
NS = 16
NW = NC * NS
L = 16
D = 128
G = 128


def _sqrt16(s):
    i = plsc.bitcast(s, jnp.int32)
    i = jnp.int32(0x5F3759DF) - (i >> 1)
    y = plsc.bitcast(i, jnp.float32)
    half_s = s * jnp.float32(0.5)
    for _ in range(3):
        y = y * (jnp.float32(1.5) - half_s * y * y)
    return s * y


def _perm(x, idx):
    return jnp.take_along_axis(x, idx, axis=0)


def _merge(x, y, k, lanes):
    m = (lanes & k) == 0
    perm = lanes ^ k
    return jnp.where(m, x, _perm(y, perm)) + jnp.where(m, _perm(x, perm), y)


def _make_body(bpw, nchunk):
    def body(h_hbm, r_hbm, t_hbm, ent_hbm, rel_hbm, out_hbm,
             idx_h, idx_r, idx_t,
             hr0, hr1, hr2, t0, t1, t2, out_v, rel_sh,
             semi, srel, sadd, st0, st1, st2):
        sid = lax.axis_index("s")
        wid = sid * NC + lax.axis_index("c")
        base = wid * bpw
        lanes = lax.iota(jnp.int32, L)

        @pl.when(sid == 0)
        def _():
            pltpu.sync_copy(rel_hbm, rel_sh)

        stg = []
        for c in range(nchunk):
            stg.append(pltpu.async_copy(
                h_hbm.at[pl.ds(base + c * G, G)], idx_h.at[c], semi))
            stg.append(pltpu.async_copy(
                r_hbm.at[pl.ds(base + c * G, G)], idx_r.at[c], semi))
            stg.append(pltpu.async_copy(
                t_hbm.at[pl.ds(base + c * G, G)], idx_t.at[c], semi))
        for d in stg:
            d.wait()
        plsc.subcore_barrier()

        hrbufs = (hr0, hr1, hr2)
        tbufs = (t0, t1, t2)
        tsems = (st0, st1, st2)

        def fire_rel(c):
            return pltpu.async_copy(rel_sh.at[idx_r.at[c]],
                                    hrbufs[c % 3], srel)

        def fire_add(c):
            return pltpu.async_copy(ent_hbm.at[idx_h.at[c]],
                                    hrbufs[c % 3], sadd, add=True)

        def fire_t(c):
            return pltpu.async_copy(ent_hbm.at[idx_t.at[c]],
                                    tbufs[c % 3], tsems[c % 3])

        d_rel = {0: fire_rel(0)}
        d_t = {0: fire_t(0)}
        d_rel[1] = fire_rel(1)
        d_t[1] = fire_t(1)
        d_rel[0].wait()
        d_add = {0: fire_add(0)}

        for c in range(nchunk):
            d_add[c].wait()
            d_t[c].wait()
            if c + 1 < nchunk:
                d_rel[c + 1].wait()
                d_add[c + 1] = fire_add(c + 1)
            if c + 2 < nchunk:
                d_rel[c + 2] = fire_rel(c + 2)
                d_t[c + 2] = fire_t(c + 2)
            rhr = hrbufs[c % 3]
            rt = tbufs[c % 3]

            def group(g, f_prev, c=c, rhr=rhr, rt=rt):
                def one_acc(row):
                    acc = jnp.zeros((L,), jnp.float32)
                    for k in range(D // L):
                        hv = rhr[row, pl.ds(k * L, L)]
                        tv = rt[row, pl.ds(k * L, L)]
                        dv = hv - tv
                        acc = acc + dv * dv
                    return acc

                perm8 = lanes ^ 8
                vs = []
                for p in range(8):
                    a = one_acc(g * 8 + p)
                    vs.append(a + _perm(a, perm8))
                for k in (4, 2, 1):
                    half = len(vs) // 2
                    vs = [_merge(x, y, k, lanes)
                          for x, y in zip(vs[:half], vs[half:])]
                f = vs[0]

                @pl.when(g % 2 == 1)
                def _():
                    res = jnp.where(lanes < 8, f_prev, f)
                    out_v[pl.ds(c * G + (g // 2) * L, L)] = _sqrt16(res)

                return f

            lax.fori_loop(0, G // 8, group, jnp.zeros((L,), jnp.float32))
        pltpu.sync_copy(out_v, out_hbm.at[pl.ds(base, bpw)])
    return body


def kernel(h, r, t, entity_emb, relation_emb):
    b = h.shape[0]
    assert b % (NW * G) == 0
    bpw = b // NW
    nchunk = bpw // G
    mesh = plsc.VectorSubcoreMesh(core_axis_name="c", subcore_axis_name="s",
                                  num_cores=NC, num_subcores=NS)
    f = pl.kernel(
        _make_body(bpw, nchunk),
        out_type=jax.ShapeDtypeStruct((b,), jnp.float32),
        mesh=mesh,
        compiler_params=pltpu.CompilerParams(
            needs_layout_passes=False,
            disable_bounds_checks=True,
            disable_semaphore_checks=True,
            skip_device_barrier=True,
        ),
        scratch_types=[
            pltpu.VMEM((nchunk, G), jnp.int32),
            pltpu.VMEM((nchunk, G), jnp.int32),
            pltpu.VMEM((nchunk, G), jnp.int32),
            pltpu.VMEM((G, D), jnp.float32),
            pltpu.VMEM((G, D), jnp.float32),
            pltpu.VMEM((G, D), jnp.float32),
            pltpu.VMEM((G, D), jnp.float32),
            pltpu.VMEM((G, D), jnp.float32),
            pltpu.VMEM((G, D), jnp.float32),
            pltpu.VMEM((bpw,), jnp.float32),
            pltpu.VMEM_SHARED(relation_emb.shape, jnp.float32),
            pltpu.SemaphoreType.DMA,
            pltpu.SemaphoreType.DMA,
            pltpu.SemaphoreType.DMA,
            pltpu.SemaphoreType.DMA,
            pltpu.SemaphoreType.DMA,
            pltpu.SemaphoreType.DMA,
        ],
    )
    return f(h.astype(jnp.int32), r.astype(jnp.int32), t.astype(jnp.int32),
             entity_emb, relation_emb)
